# baseline (speedup 1.0000x reference)
import jax
import jax.numpy as jnp
from jax import lax
from jax.experimental import pallas as pl
from jax.experimental.pallas import tpu as pltpu
from jax.experimental.pallas import tpu_sc as plsc

B, T, D = 16, 4096, 300
NC, NS = 2, 16
NW = NC * NS
DSC = 50
DBLK = 50
LANES = 16
NVR = T // LANES


def _tc_body(x_ref, xl_ref, o_ref):
    xl = xl_ref[...][:, 0:1]
    tio = lax.broadcasted_iota(jnp.int32, (B, T), 1)
    mask = tio < xl
    o_ref[...] = jnp.where(mask[None, :, :], x_ref[...], 0.0)


def _masked_copy_tc(xt, xl2d):
    ntc = D - DSC
    return pl.pallas_call(
        _tc_body,
        grid=(ntc // DBLK,),
        in_specs=[
            pl.BlockSpec((DBLK, B, T), lambda i: (i + DSC // DBLK, 0, 0)),
            pl.BlockSpec((B, 128), lambda i: (0, 0)),
        ],
        out_specs=pl.BlockSpec((DBLK, B, T), lambda i: (i, 0, 0)),
        out_shape=jax.ShapeDtypeStruct((ntc, B, T), jnp.float32),
    )(xt, xl2d)


def _sc_body(x_hbm, xlen_hbm, out_hbm, vbuf, xlen_v, in_sem, out_sem):
    w = lax.axis_index("s") * NC + lax.axis_index("c")
    pltpu.sync_copy(xlen_hbm, xlen_v)
    xlen_vec = xlen_v[...]
    zeros16 = jnp.zeros((LANES,), jnp.float32)

    for k in range((DSC + NW - 1) // NW):
        d = w + k * NW

        @pl.when(d < DSC)
        def _row():
            pltpu.async_copy(x_hbm.at[d], vbuf, in_sem)
            pltpu.make_async_copy(x_hbm.at[d], vbuf, in_sem).wait()

            for b in range(B):
                xl = xlen_vec[b]
                j0 = xl // LANES

                @pl.when(xl < T)
                def _fix_boundary():
                    vec = vbuf[b, pl.ds(j0 * LANES, LANES)]
                    pos = j0 * LANES + lax.iota(jnp.int32, LANES)
                    vbuf[b, pl.ds(j0 * LANES, LANES)] = jnp.where(
                        pos < xl, vec, 0.0)

                def zero_tail(j, carry):
                    vbuf[b, pl.ds(j * LANES, LANES)] = zeros16
                    return carry

                lax.fori_loop(j0 + 1, NVR, zero_tail, 0)

            pltpu.async_copy(vbuf, out_hbm.at[d], out_sem)
            pltpu.make_async_copy(vbuf, out_hbm.at[d], out_sem).wait()


def _masked_copy_sc(xt, x_len):
    run = pl.kernel(
        _sc_body,
        mesh=plsc.VectorSubcoreMesh(core_axis_name="c", subcore_axis_name="s"),
        out_type=jax.ShapeDtypeStruct((DSC, B, T), jnp.float32),
        scratch_types=[
            pltpu.VMEM((B, T), jnp.float32),
            pltpu.VMEM((B,), jnp.int32),
            pltpu.SemaphoreType.DMA,
            pltpu.SemaphoreType.DMA,
        ],
        compiler_params=pltpu.CompilerParams(use_tc_tiling_on_sc=True),
    )
    return run(xt, x_len)


def kernel(x, x_len):
    xl = x_len.astype(jnp.int32)
    xt = lax.transpose(x, (2, 0, 1))
    xl2d = jnp.broadcast_to(xl[:, None], (B, 128))
    sc_part = _masked_copy_sc(xt, xl)
    tc_part = _masked_copy_tc(xt, xl2d)
    out_t = lax.concatenate([sc_part, tc_part], 0)
    return lax.transpose(out_t, (1, 2, 0))


# speedup vs baseline: 1.0078x; 1.0078x over previous
import jax
import jax.numpy as jnp
from jax import lax
from jax.experimental import pallas as pl
from jax.experimental.pallas import tpu as pltpu
from jax.experimental.pallas import tpu_sc as plsc

B, T, D = 16, 4096, 300
NC, NS = 2, 16
NW = NC * NS
DSC = 30
DBLK = 30
LANES = 16
NVR = T // LANES


def _tc_body(x_ref, xl_ref, o_ref):
    xl = xl_ref[...][:, 0:1]
    tio = lax.broadcasted_iota(jnp.int32, (B, T), 1)
    mask = tio < xl
    o_ref[...] = jnp.where(mask[None, :, :], x_ref[...], 0.0)


def _masked_copy_tc(xt, xl2d):
    ntc = D - DSC
    return pl.pallas_call(
        _tc_body,
        grid=(ntc // DBLK,),
        in_specs=[
            pl.BlockSpec((DBLK, B, T), lambda i: (i + DSC // DBLK, 0, 0)),
            pl.BlockSpec((B, 128), lambda i: (0, 0)),
        ],
        out_specs=pl.BlockSpec((DBLK, B, T), lambda i: (i, 0, 0)),
        out_shape=jax.ShapeDtypeStruct((ntc, B, T), jnp.float32),
    )(xt, xl2d)


def _sc_body(x_hbm, xlen_hbm, out_hbm, vbuf, xlen_v, in_sem, out_sem):
    w = lax.axis_index("s") * NC + lax.axis_index("c")
    pltpu.sync_copy(xlen_hbm, xlen_v)
    xlen_vec = xlen_v[...]
    zeros16 = jnp.zeros((LANES,), jnp.float32)

    for k in range((DSC + NW - 1) // NW):
        d = w + k * NW

        @pl.when(d < DSC)
        def _row():
            pltpu.async_copy(x_hbm.at[d], vbuf, in_sem)
            pltpu.make_async_copy(x_hbm.at[d], vbuf, in_sem).wait()

            for b in range(B):
                xl = xlen_vec[b]
                jb0 = xl // (8 * LANES)

                @pl.when(xl < T)
                def _fix_boundary_group():
                    for u in range(8):
                        j = jb0 * 8 + u
                        vec = vbuf[b, pl.ds(j * LANES, LANES)]
                        pos = j * LANES + lax.iota(jnp.int32, LANES)
                        vbuf[b, pl.ds(j * LANES, LANES)] = jnp.where(
                            pos < xl, vec, 0.0)

                def zero_tail(jb, carry):
                    for u in range(8):
                        vbuf[b, pl.ds((jb * 8 + u) * LANES, LANES)] = zeros16
                    return carry

                lax.fori_loop(jb0 + 1, NVR // 8, zero_tail, 0)

            pltpu.async_copy(vbuf, out_hbm.at[d], out_sem)
            pltpu.make_async_copy(vbuf, out_hbm.at[d], out_sem).wait()


def _masked_copy_sc(xt, x_len):
    run = pl.kernel(
        _sc_body,
        mesh=plsc.VectorSubcoreMesh(core_axis_name="c", subcore_axis_name="s"),
        out_type=jax.ShapeDtypeStruct((DSC, B, T), jnp.float32),
        scratch_types=[
            pltpu.VMEM((B, T), jnp.float32),
            pltpu.VMEM((B,), jnp.int32),
            pltpu.SemaphoreType.DMA,
            pltpu.SemaphoreType.DMA,
        ],
        compiler_params=pltpu.CompilerParams(use_tc_tiling_on_sc=True),
    )
    return run(xt, x_len)


def kernel(x, x_len):
    xl = x_len.astype(jnp.int32)
    xt = lax.transpose(x, (2, 0, 1))
    xl2d = jnp.broadcast_to(xl[:, None], (B, 128))
    sc_part = _masked_copy_sc(xt, xl)
    tc_part = _masked_copy_tc(xt, xl2d)
    out_t = lax.concatenate([sc_part, tc_part], 0)
    return lax.transpose(out_t, (1, 2, 0))


# speedup vs baseline: 2.3537x; 2.3354x over previous
import jax
import jax.numpy as jnp
from jax import lax
from jax.experimental import pallas as pl
from jax.experimental.pallas import tpu as pltpu

B, T, D = 16, 4096, 300
DBLK = 60


def _tc_body(x_ref, xl_ref, o_ref):
    xl = xl_ref[...][:, 0:1]
    tio = lax.broadcasted_iota(jnp.int32, (B, T), 1)
    mask = tio < xl
    o_ref[...] = jnp.where(mask[None, :, :], x_ref[...], 0.0)


def _masked_copy_tc(xt, xl2d):
    return pl.pallas_call(
        _tc_body,
        grid=(D // DBLK,),
        in_specs=[
            pl.BlockSpec((DBLK, B, T), lambda i: (i, 0, 0)),
            pl.BlockSpec((B, 128), lambda i: (0, 0)),
        ],
        out_specs=pl.BlockSpec((DBLK, B, T), lambda i: (i, 0, 0)),
        out_shape=jax.ShapeDtypeStruct((D, B, T), jnp.float32),
        compiler_params=pltpu.CompilerParams(
            vmem_limit_bytes=67043328,
        ),
    )(xt, xl2d)


def kernel(x, x_len):
    xt = lax.transpose(x, (2, 0, 1))
    xl2d = jnp.broadcast_to(x_len.astype(jnp.int32)[:, None], (B, 128))
    out_t = _masked_copy_tc(xt, xl2d)
    return lax.transpose(out_t, (1, 2, 0))


# speedup vs baseline: 2.3594x; 1.0024x over previous
import jax
import jax.numpy as jnp
from jax import lax
from jax.experimental import pallas as pl
from jax.experimental.pallas import tpu as pltpu

B, T, D = 16, 4096, 300
DBLK = 50


def _tc_body(x_ref, xl_ref, o_ref):
    xl = xl_ref[...][:, 0:1]
    tio = lax.broadcasted_iota(jnp.int32, (B, T), 1)
    mask = tio < xl
    o_ref[...] = jnp.where(mask[None, :, :], x_ref[...], 0.0)


def _masked_copy_tc(xt, xl2d):
    return pl.pallas_call(
        _tc_body,
        grid=(D // DBLK,),
        in_specs=[
            pl.BlockSpec((DBLK, B, T), lambda i: (i, 0, 0)),
            pl.BlockSpec((B, 128), lambda i: (0, 0)),
        ],
        out_specs=pl.BlockSpec((DBLK, B, T), lambda i: (i, 0, 0)),
        out_shape=jax.ShapeDtypeStruct((D, B, T), jnp.float32),
    )(xt, xl2d)


def kernel(x, x_len):
    xt = lax.transpose(x, (2, 0, 1))
    xl2d = jnp.broadcast_to(x_len.astype(jnp.int32)[:, None], (B, 128))
    out_t = _masked_copy_tc(xt, xl2d)
    return lax.transpose(out_t, (1, 2, 0))
